# baseline (speedup 1.0000x reference)
import jax
import jax.numpy as jnp
from jax.experimental import pallas as pl

HID = 1024
NK = 256
K = 8
H = 8


def _bn_body(x_ref, g_ref, b_ref, o_ref):
    x = x_ref[...]
    mean = jnp.mean(x, axis=0, keepdims=True)
    var = jnp.mean((x - mean) ** 2, axis=0, keepdims=True)
    o_ref[...] = (x - mean) / jnp.sqrt(var + 1e-5) * g_ref[...] + b_ref[...]


def kernel(inputs, bn_gamma, bn_beta, W_q, keys, down_w, up_w):
    b, n, d = inputs.shape
    x = inputs.reshape(b * n, d)
    xn = pl.pallas_call(
        _bn_body,
        out_shape=jax.ShapeDtypeStruct((b * n, d), jnp.float32),
    )(x, bn_gamma.reshape(1, d), bn_beta.reshape(1, d))
    q = (xn @ W_q.T).reshape(b, n, 2, H, 128)
    q = jnp.transpose(q, (2, 0, 1, 3, 4))
    sim = jnp.einsum('pbnhd,hkpd->pbnhk', q, keys)
    s, idx = jax.lax.top_k(sim, K)
    scores_x, scores_y = s[0], s[1]
    indices_x, indices_y = idx[0], idx[1]
    all_scores = (scores_x[..., :, None] + scores_y[..., None, :]).reshape(b, n, H, K * K)
    all_indices = (indices_x[..., :, None] * NK + indices_y[..., None, :]).reshape(b, n, H, K * K)
    scores, pk_indices = jax.lax.top_k(all_scores, K)
    indices = jnp.take_along_axis(all_indices, pk_indices, axis=-1)
    weights_down = jnp.take(down_w, indices, axis=0)
    outputs = jnp.einsum('bnd,bnhkd->bnhk', inputs, weights_down)
    outputs = jax.nn.sigmoid(scores) * jax.nn.gelu(outputs, approximate=False)
    weights_up = jnp.take(up_w, indices, axis=0)
    outputs = jnp.einsum('bnhk,bnhkd->bnd', outputs, weights_up)
    return outputs


# speedup vs baseline: 3.4700x; 3.4700x over previous
import functools

import jax
import jax.numpy as jnp
from jax import lax
from jax.experimental import pallas as pl
from jax.experimental.pallas import tpu as pltpu
from jax.experimental.pallas import tpu_sc as plsc

HIDDEN = 1024
KDIM = 128
TOPK = 8
HEADS = 8
NKEYS = 256
NTOK = 2048
TOKB = 256
NEG_INF = float("-inf")

SC_CORES = 2
SC_SUBCORES = 16
SC_WORKERS = SC_CORES * SC_SUBCORES
TOK_PER_W = NTOK // SC_WORKERS
CHUNK = 16
NCHUNK = (HEADS * TOPK) // CHUNK
DCH = HIDDEN // 16


def _stats_body(x_ref, mean_ref, var_ref):
    x = x_ref[...]
    mean = jnp.mean(x, axis=0, keepdims=True)
    var = jnp.mean((x - mean) ** 2, axis=0, keepdims=True)
    mean_ref[...] = jnp.broadcast_to(mean, (8, HIDDEN))
    var_ref[...] = jnp.broadcast_to(var, (8, HIDDEN))


def _bn_stats(x):
    return pl.pallas_call(
        _stats_body,
        out_shape=(
            jax.ShapeDtypeStruct((8, HIDDEN), jnp.float32),
            jax.ShapeDtypeStruct((8, HIDDEN), jnp.float32),
        ),
    )(x)


def _topk_exact(work, k, iota):
    ncand = work.shape[1]
    ms, ams = [], []
    for _ in range(k):
        m = jnp.max(work, axis=1, keepdims=True)
        am = jnp.min(jnp.where(work == m, iota, ncand), axis=1, keepdims=True)
        ms.append(m)
        ams.append(am)
        work = jnp.where(iota == am, NEG_INF, work)
    return jnp.concatenate(ms, axis=1), jnp.concatenate(ams, axis=1)


def _routing_body(x_ref, sb_ref, wq_ref, kt_ref, sig_ref, eid_ref):
    mean = sb_ref[0:1, :]
    var = sb_ref[1:2, :]
    gamma = sb_ref[2:3, :]
    beta = sb_ref[3:4, :]
    xn = (x_ref[...] - mean) / jnp.sqrt(var + 1e-5) * gamma + beta
    q = None
    for kb in range(HIDDEN // 256):
        part = lax.dot_general(
            xn[:, kb * 256:(kb + 1) * 256],
            wq_ref[kb * 256:(kb + 1) * 256, :],
            (((1,), (0,)), ((), ())), preferred_element_type=jnp.float32)
        q = part if q is None else q + part
    iota256 = lax.broadcasted_iota(jnp.int32, (TOKB, NKEYS), 1)
    vals = []
    for ph in range(2 * HEADS):
        qs = q[:, ph * KDIM:(ph + 1) * KDIM]
        sim = lax.dot_general(qs, kt_ref[ph], (((1,), (1,)), ((), ())),
                              preferred_element_type=jnp.float32)
        vals.append(_topk_exact(sim, TOPK, iota256))
    iota64 = lax.broadcasted_iota(jnp.int32, (TOKB, TOPK * TOPK), 1)
    sig_cols = []
    eid_cols = []
    for h in range(HEADS):
        vx, ix = vals[h]
        vy, iy = vals[HEADS + h]
        cand_s = jnp.concatenate(
            [vx[:, i:i + 1] + vy for i in range(TOPK)], axis=1)
        cand_e = jnp.concatenate(
            [ix[:, i:i + 1] * NKEYS + iy for i in range(TOPK)], axis=1)
        sval, pos = _topk_exact(cand_s, TOPK, iota64)
        sig_cols.append(1.0 / (1.0 + jnp.exp(-sval)))
        eid_k = []
        for k in range(TOPK):
            sel = jnp.where(iota64 == pos[:, k:k + 1], cand_e, 0)
            eid_k.append(jnp.sum(sel, axis=1, keepdims=True))
        eid_cols.append(jnp.concatenate(eid_k, axis=1))
    sig_ref[...] = jnp.concatenate(sig_cols, axis=1)
    eid_ref[...] = jnp.concatenate(eid_cols, axis=1)


def _routing(x, scalebias, W_q, kt):
    nblk = NTOK // TOKB
    return pl.pallas_call(
        _routing_body,
        grid=(nblk,),
        in_specs=[
            pl.BlockSpec((TOKB, HIDDEN), lambda i: (i, 0)),
            pl.BlockSpec((8, HIDDEN), lambda i: (0, 0)),
            pl.BlockSpec((HIDDEN, 2 * HEADS * KDIM), lambda i: (0, 0)),
            pl.BlockSpec((2 * HEADS, NKEYS, KDIM), lambda i: (0, 0, 0)),
        ],
        out_specs=(
            pl.BlockSpec((TOKB, HEADS * TOPK), lambda i: (i, 0)),
            pl.BlockSpec((TOKB, HEADS * TOPK), lambda i: (i, 0)),
        ),
        out_shape=(
            jax.ShapeDtypeStruct((NTOK, HEADS * TOPK), jnp.float32),
            jax.ShapeDtypeStruct((NTOK, HEADS * TOPK), jnp.int32),
        ),
    )(x, scalebias, W_q, kt)


def _gelu16(v):
    ax = jnp.abs(v) * jnp.float32(0.7071067811865476)
    t = 1.0 / (1.0 + jnp.float32(0.3275911) * ax)
    poly = ((((jnp.float32(1.061405429) * t + jnp.float32(-1.453152027)) * t
              + jnp.float32(1.421413741)) * t + jnp.float32(-0.284496736)) * t
            + jnp.float32(0.254829592)) * t
    erf = 1.0 - poly * jnp.exp(-ax * ax)
    erf = jnp.where(v >= 0.0, erf, -erf)
    return 0.5 * v * (1.0 + erf)


def _expert_body(x_hbm, sig_hbm, idx_hbm, down_hbm, up_hbm, out_hbm,
                 xv, sigv, idxv, downb, upb, outv, sem_d, sem_u):
    wid = lax.axis_index("s") * SC_CORES + lax.axis_index("c")
    base = wid * TOK_PER_W
    zeros16 = jnp.zeros((16,), jnp.float32)
    iota16 = lax.iota(jnp.int32, 16)

    def token_body(t, _):
        n = base + t
        pltpu.sync_copy(x_hbm.at[n], xv)
        pltpu.sync_copy(sig_hbm.at[n], sigv)
        pltpu.sync_copy(idx_hbm.at[n], idxv)
        for d in range(DCH):
            outv[pl.ds(16 * d, 16)] = zeros16

        def chunk_body(c, _):
            idx_vec = idxv[pl.ds(c * CHUNK, CHUNK)]
            cp_d = pltpu.async_copy(down_hbm.at[idx_vec], downb, sem_d)
            cp_u = pltpu.async_copy(up_hbm.at[idx_vec], upb, sem_u)
            cp_d.wait()
            dots = zeros16
            for g in range(2):
                def dot_step(db, accs):
                    new = list(accs)
                    for dd in range(8):
                        d = db * 8 + dd
                        s = pl.ds(16 * d, 16)
                        xc = xv[s]
                        for r in range(8):
                            new[r] = new[r] + xc * downb[g * 8 + r, s]
                    return tuple(new)
                accs = lax.fori_loop(0, 8, dot_step, (zeros16,) * 8)
                for r in range(8):
                    a = accs[r]
                    for sh in (8, 4, 2, 1):
                        a = a + jnp.take(a, (iota16 + sh) & 15)
                    dots = dots + jnp.where(iota16 == (g * 8 + r), a, 0.0)
            w = sigv[pl.ds(c * CHUNK, CHUNK)] * _gelu16(dots)
            cp_u.wait()
            for r in range(CHUNK):
                wr = lax.gather(
                    w, jnp.full((16, 1), r, jnp.int32),
                    lax.GatherDimensionNumbers(
                        offset_dims=(), collapsed_slice_dims=(0,),
                        start_index_map=(0,)),
                    (1,), mode=lax.GatherScatterMode.PROMISE_IN_BOUNDS)
                def up_step(db, _, r=r, wr=wr):
                    for dd in range(8):
                        s = pl.ds(16 * (db * 8 + dd), 16)
                        plsc.addupdate(outv.at[s], wr * upb[r, s])
                    return 0
                lax.fori_loop(0, 8, up_step, 0)
            return 0

        lax.fori_loop(0, NCHUNK, chunk_body, 0)
        pltpu.sync_copy(outv, out_hbm.at[n])
        return 0

    lax.fori_loop(0, TOK_PER_W, token_body, 0)


def _expert_combine(x, sigT, eidT, down_w, up_w):
    mesh = plsc.VectorSubcoreMesh(
        core_axis_name="c", subcore_axis_name="s",
        num_cores=SC_CORES, num_subcores=SC_SUBCORES)
    f = functools.partial(
        pl.kernel,
        out_type=jax.ShapeDtypeStruct((NTOK, HIDDEN), jnp.float32),
        mesh=mesh,
        scratch_types=[
            pltpu.VMEM((HIDDEN,), jnp.float32),
            pltpu.VMEM((HEADS * TOPK,), jnp.float32),
            pltpu.VMEM((HEADS * TOPK,), jnp.int32),
            pltpu.VMEM((CHUNK, HIDDEN), jnp.float32),
            pltpu.VMEM((CHUNK, HIDDEN), jnp.float32),
            pltpu.VMEM((HIDDEN,), jnp.float32),
            pltpu.SemaphoreType.DMA,
            pltpu.SemaphoreType.DMA,
        ],
    )(_expert_body)
    return f(x, sigT, eidT, down_w, up_w)


def kernel(inputs, bn_gamma, bn_beta, W_q, keys, down_w, up_w):
    b, n, d = inputs.shape
    x = inputs.reshape(n, d)
    mean8, var8 = _bn_stats(x)
    stats = jnp.tile(
        jnp.stack([mean8[0], var8[0], bn_gamma, bn_beta], axis=0), (2, 1))
    kt = jnp.transpose(keys, (2, 0, 1, 3)).reshape(2 * HEADS, NKEYS, KDIM)
    sigT, eidT = _routing(x, stats, W_q.T, kt)
    out = _expert_combine(x, sigT, eidT, down_w, up_w)
    return out.reshape(b, n, d)


# speedup vs baseline: 4.3442x; 1.2520x over previous
import functools

import jax
import jax.numpy as jnp
from jax import lax
from jax.experimental import pallas as pl
from jax.experimental.pallas import tpu as pltpu
from jax.experimental.pallas import tpu_sc as plsc

HIDDEN = 1024
KDIM = 128
TOPK = 8
HEADS = 8
NKEYS = 256
NTOK = 2048
TOKB = 256
NEG_INF = float("-inf")

SC_CORES = 2
SC_SUBCORES = 16
SC_WORKERS = SC_CORES * SC_SUBCORES
TOK_PER_W = NTOK // SC_WORKERS
CHUNK = 16
NCHUNK = (HEADS * TOPK) // CHUNK
DCH = HIDDEN // 16


def _stats_body(x_ref, mean_ref, var_ref):
    x = x_ref[...]
    mean = jnp.mean(x, axis=0, keepdims=True)
    var = jnp.mean((x - mean) ** 2, axis=0, keepdims=True)
    mean_ref[...] = jnp.broadcast_to(mean, (8, HIDDEN))
    var_ref[...] = jnp.broadcast_to(var, (8, HIDDEN))


def _bn_stats(x):
    return pl.pallas_call(
        _stats_body,
        out_shape=(
            jax.ShapeDtypeStruct((8, HIDDEN), jnp.float32),
            jax.ShapeDtypeStruct((8, HIDDEN), jnp.float32),
        ),
    )(x)


def _topk_exact(work, k, iota):
    ncand = work.shape[1]
    ms, ams = [], []
    for _ in range(k):
        m = jnp.max(work, axis=1, keepdims=True)
        am = jnp.min(jnp.where(work == m, iota, ncand), axis=1, keepdims=True)
        ms.append(m)
        ams.append(am)
        work = jnp.where(iota == am, NEG_INF, work)
    return jnp.concatenate(ms, axis=1), jnp.concatenate(ams, axis=1)


def _routing_body(x_ref, sb_ref, wq_ref, kt_ref, sig_ref, eid_ref):
    mean = sb_ref[0:1, :]
    var = sb_ref[1:2, :]
    gamma = sb_ref[2:3, :]
    beta = sb_ref[3:4, :]
    xn = (x_ref[...] - mean) / jnp.sqrt(var + 1e-5) * gamma + beta
    q = None
    for kb in range(HIDDEN // 256):
        part = lax.dot_general(
            xn[:, kb * 256:(kb + 1) * 256],
            wq_ref[kb * 256:(kb + 1) * 256, :],
            (((1,), (0,)), ((), ())), preferred_element_type=jnp.float32)
        q = part if q is None else q + part
    iota256 = lax.broadcasted_iota(jnp.int32, (TOKB, NKEYS), 1)
    vals = []
    for ph in range(2 * HEADS):
        qs = q[:, ph * KDIM:(ph + 1) * KDIM]
        sim = lax.dot_general(qs, kt_ref[ph], (((1,), (1,)), ((), ())),
                              preferred_element_type=jnp.float32)
        vals.append(_topk_exact(sim, TOPK, iota256))
    iota64 = lax.broadcasted_iota(jnp.int32, (TOKB, TOPK * TOPK), 1)
    sig_cols = []
    eid_cols = []
    for h in range(HEADS):
        vx, ix = vals[h]
        vy, iy = vals[HEADS + h]
        cand_s = jnp.concatenate(
            [vx[:, i:i + 1] + vy for i in range(TOPK)], axis=1)
        cand_e = jnp.concatenate(
            [ix[:, i:i + 1] * NKEYS + iy for i in range(TOPK)], axis=1)
        sval, pos = _topk_exact(cand_s, TOPK, iota64)
        sig_cols.append(1.0 / (1.0 + jnp.exp(-sval)))
        eid_k = []
        for k in range(TOPK):
            sel = jnp.where(iota64 == pos[:, k:k + 1], cand_e, 0)
            eid_k.append(jnp.sum(sel, axis=1, keepdims=True))
        eid_cols.append(jnp.concatenate(eid_k, axis=1))
    sig_ref[...] = jnp.concatenate(sig_cols, axis=1)
    eid_ref[...] = jnp.concatenate(eid_cols, axis=1)


def _routing(x, scalebias, W_q, kt):
    nblk = NTOK // TOKB
    return pl.pallas_call(
        _routing_body,
        grid=(nblk,),
        in_specs=[
            pl.BlockSpec((TOKB, HIDDEN), lambda i: (i, 0)),
            pl.BlockSpec((8, HIDDEN), lambda i: (0, 0)),
            pl.BlockSpec((HIDDEN, 2 * HEADS * KDIM), lambda i: (0, 0)),
            pl.BlockSpec((2 * HEADS, NKEYS, KDIM), lambda i: (0, 0, 0)),
        ],
        out_specs=(
            pl.BlockSpec((TOKB, HEADS * TOPK), lambda i: (i, 0)),
            pl.BlockSpec((TOKB, HEADS * TOPK), lambda i: (i, 0)),
        ),
        out_shape=(
            jax.ShapeDtypeStruct((NTOK, HEADS * TOPK), jnp.float32),
            jax.ShapeDtypeStruct((NTOK, HEADS * TOPK), jnp.int32),
        ),
    )(x, scalebias, W_q, kt)


def _gelu16(v):
    ax = jnp.abs(v) * jnp.float32(0.7071067811865476)
    t = 1.0 / (1.0 + jnp.float32(0.3275911) * ax)
    poly = ((((jnp.float32(1.061405429) * t + jnp.float32(-1.453152027)) * t
              + jnp.float32(1.421413741)) * t + jnp.float32(-0.284496736)) * t
            + jnp.float32(0.254829592)) * t
    erf = 1.0 - poly * jnp.exp(-ax * ax)
    erf = jnp.where(v >= 0.0, erf, -erf)
    return 0.5 * v * (1.0 + erf)


def _expert_body(x_hbm, sig_hbm, idx_hbm, down_hbm, up_hbm, out_hbm,
                 xv, sigv, idxv, outv, dbuf, ubuf,
                 sem_d0, sem_d1, sem_u0, sem_u1,
                 sem_pf0, sem_pf1, sem_w0, sem_w1):
    wid = lax.axis_index("s") * SC_CORES + lax.axis_index("c")
    base = wid * TOK_PER_W
    zeros16 = jnp.zeros((16,), jnp.float32)
    iota16 = lax.iota(jnp.int32, 16)
    sem_d = (sem_d0, sem_d1)
    sem_u = (sem_u0, sem_u1)
    sem_pf = (sem_pf0, sem_pf1)
    sem_w = (sem_w0, sem_w1)

    def issue_pf(tok, par):
        n = base + tok
        pltpu.async_copy(x_hbm.at[n], xv.at[par], sem_pf[par])
        pltpu.async_copy(sig_hbm.at[n], sigv.at[par], sem_pf[par])
        pltpu.async_copy(idx_hbm.at[n], idxv.at[par], sem_pf[par])

    def wait_pf(tok, par):
        n = base + tok
        pltpu.make_async_copy(x_hbm.at[n], xv.at[par], sem_pf[par]).wait()
        pltpu.make_async_copy(sig_hbm.at[n], sigv.at[par], sem_pf[par]).wait()
        pltpu.make_async_copy(idx_hbm.at[n], idxv.at[par], sem_pf[par]).wait()

    def issue_gather(par, c, b):
        iv = idxv[par, pl.ds(c * CHUNK, CHUNK)]
        pltpu.async_copy(down_hbm.at[iv], dbuf.at[b], sem_d[b])
        pltpu.async_copy(up_hbm.at[iv], ubuf.at[b], sem_u[b])

    def compute_chunk(par, c, b):
        iv = idxv[par, pl.ds(c * CHUNK, CHUNK)]
        pltpu.make_async_copy(down_hbm.at[iv], dbuf.at[b], sem_d[b]).wait()
        dots = zeros16
        for g in range(2):
            def dot_step(db, accs, g=g):
                new = list(accs)
                for dd in range(8):
                    s = pl.ds(16 * (db * 8 + dd), 16)
                    xc = xv[par, s]
                    for r in range(8):
                        new[r] = new[r] + xc * dbuf[b, g * 8 + r, s]
                return tuple(new)
            accs = lax.fori_loop(0, 8, dot_step, (zeros16,) * 8)
            for r in range(8):
                a = accs[r]
                for sh in (8, 4, 2, 1):
                    a = a + jnp.take(a, (iota16 + sh) & 15)
                dots = dots + jnp.where(iota16 == (g * 8 + r), a, 0.0)
        w = sigv[par, pl.ds(c * CHUNK, CHUNK)] * _gelu16(dots)
        pltpu.make_async_copy(up_hbm.at[iv], ubuf.at[b], sem_u[b]).wait()

        def up_row(r, _):
            wr = jnp.take(w, jnp.full((16,), 0, jnp.int32) + r)
            def up_step(db, _):
                for dd in range(8):
                    s = pl.ds(16 * (db * 8 + dd), 16)
                    plsc.addupdate(outv.at[par, s], wr * ubuf[b, r, s])
                return 0
            lax.fori_loop(0, 8, up_step, 0)
            return 0
        lax.fori_loop(0, CHUNK, up_row, 0)

    def zero_out(par):
        for d in range(DCH):
            outv[par, pl.ds(16 * d, 16)] = zeros16

    def issue_write(tok, par):
        pltpu.async_copy(outv.at[par], out_hbm.at[base + tok], sem_w[par])

    def wait_write(tok, par):
        pltpu.make_async_copy(outv.at[par], out_hbm.at[base + tok],
                              sem_w[par]).wait()

    issue_pf(0, 0)
    wait_pf(0, 0)
    issue_pf(1, 1)
    issue_gather(0, 0, 0)
    issue_gather(0, 1, 1)

    npair = TOK_PER_W // 2

    def pair_body(i, _):
        ta = 2 * i
        tb = 2 * i + 1

        @pl.when(i > 0)
        def _():
            wait_write(ta - 2, 0)
        zero_out(0)
        compute_chunk(0, 0, 0)
        issue_gather(0, 2, 0)
        compute_chunk(0, 1, 1)
        issue_gather(0, 3, 1)
        compute_chunk(0, 2, 0)
        wait_pf(tb, 1)
        issue_gather(1, 0, 0)
        compute_chunk(0, 3, 1)
        issue_write(ta, 0)

        @pl.when(i < npair - 1)
        def _():
            issue_pf(ta + 2, 0)
        issue_gather(1, 1, 1)

        @pl.when(i > 0)
        def _():
            wait_write(tb - 2, 1)
        zero_out(1)
        compute_chunk(1, 0, 0)
        issue_gather(1, 2, 0)
        compute_chunk(1, 1, 1)
        issue_gather(1, 3, 1)
        compute_chunk(1, 2, 0)

        @pl.when(i < npair - 1)
        def _():
            wait_pf(ta + 2, 0)
            issue_gather(0, 0, 0)
        compute_chunk(1, 3, 1)
        issue_write(tb, 1)

        @pl.when(i < npair - 1)
        def _():
            issue_pf(tb + 2, 1)
            issue_gather(0, 1, 1)
        return 0

    lax.fori_loop(0, npair, pair_body, 0)
    wait_write(TOK_PER_W - 2, 0)
    wait_write(TOK_PER_W - 1, 1)


def _expert_combine(x, sigT, eidT, down_w, up_w):
    mesh = plsc.VectorSubcoreMesh(
        core_axis_name="c", subcore_axis_name="s",
        num_cores=SC_CORES, num_subcores=SC_SUBCORES)
    f = functools.partial(
        pl.kernel,
        out_type=jax.ShapeDtypeStruct((NTOK, HIDDEN), jnp.float32),
        mesh=mesh,
        scratch_types=[
            pltpu.VMEM((2, HIDDEN), jnp.float32),
            pltpu.VMEM((2, HEADS * TOPK), jnp.float32),
            pltpu.VMEM((2, HEADS * TOPK), jnp.int32),
            pltpu.VMEM((2, HIDDEN), jnp.float32),
            pltpu.VMEM((2, CHUNK, HIDDEN), jnp.float32),
            pltpu.VMEM((2, CHUNK, HIDDEN), jnp.float32),
            pltpu.SemaphoreType.DMA,
            pltpu.SemaphoreType.DMA,
            pltpu.SemaphoreType.DMA,
            pltpu.SemaphoreType.DMA,
            pltpu.SemaphoreType.DMA,
            pltpu.SemaphoreType.DMA,
            pltpu.SemaphoreType.DMA,
            pltpu.SemaphoreType.DMA,
        ],
    )(_expert_body)
    return f(x, sigT, eidT, down_w, up_w)


def kernel(inputs, bn_gamma, bn_beta, W_q, keys, down_w, up_w):
    b, n, d = inputs.shape
    x = inputs.reshape(n, d)
    mean8, var8 = _bn_stats(x)
    stats = jnp.tile(
        jnp.stack([mean8[0], var8[0], bn_gamma, bn_beta], axis=0), (2, 1))
    kt = jnp.transpose(keys, (2, 0, 1, 3)).reshape(2 * HEADS, NKEYS, KDIM)
    sigT, eidT = _routing(x, stats, W_q.T, kt)
    out = _expert_combine(x, sigT, eidT, down_w, up_w)
    return out.reshape(b, n, d)
